# baseline (speedup 1.0000x reference)
import functools

import jax
import jax.numpy as jnp
from jax import lax
from jax.experimental import pallas as pl
from jax.experimental.pallas import tpu as pltpu
from jax.experimental.pallas import tpu_sc as plsc

EMBED_DIM = 64
IDX_CHUNK = 128


def _make_gather_kernel(batch, num_workers, chunks_per_worker):
    rows_per_worker = chunks_per_worker * IDX_CHUNK
    mesh = plsc.VectorSubcoreMesh(core_axis_name="c", subcore_axis_name="s")

    @functools.partial(
        pl.kernel,
        out_type=(
            jax.ShapeDtypeStruct((batch, EMBED_DIM), jnp.float32),
            jax.ShapeDtypeStruct((batch, EMBED_DIM), jnp.float32),
        ),
        mesh=mesh,
        compiler_params=pltpu.CompilerParams(use_tc_tiling_on_sc=False),
        scratch_types=[
            pltpu.VMEM((chunks_per_worker, IDX_CHUNK), jnp.int32),
            pltpu.VMEM((chunks_per_worker, IDX_CHUNK), jnp.int32),
            pltpu.VMEM((rows_per_worker, EMBED_DIM), jnp.float32),
            pltpu.VMEM((rows_per_worker, EMBED_DIM), jnp.float32),
            pltpu.SemaphoreType.DMA,
        ],
    )
    def gather_kernel(uidx_hbm, aidx_hbm, utab_hbm, atab_hbm,
                      uout_hbm, aout_hbm,
                      uidx_v, aidx_v, urows_v, arows_v, sem):
        wid = lax.axis_index("s") * 2 + lax.axis_index("c")
        crow = wid * chunks_per_worker
        pltpu.sync_copy(uidx_hbm.at[pl.ds(crow, chunks_per_worker)], uidx_v)
        pltpu.sync_copy(aidx_hbm.at[pl.ds(crow, chunks_per_worker)], aidx_v)
        copies = []
        for j in range(chunks_per_worker):
            copies.append(pltpu.async_copy(
                utab_hbm.at[uidx_v.at[j]],
                urows_v.at[pl.ds(j * IDX_CHUNK, IDX_CHUNK)], sem))
            copies.append(pltpu.async_copy(
                atab_hbm.at[aidx_v.at[j]],
                arows_v.at[pl.ds(j * IDX_CHUNK, IDX_CHUNK)], sem))
        for c in copies:
            c.wait()
        base = wid * rows_per_worker
        pltpu.sync_copy(urows_v, uout_hbm.at[pl.ds(base, rows_per_worker)])
        pltpu.sync_copy(arows_v, aout_hbm.at[pl.ds(base, rows_per_worker)])

    return gather_kernel


def _mlp_body(uref, aref, w1u_ref, w1a_ref, b1_ref, w2_ref, b2_ref, w3_ref,
              out_ref):
    h1 = jnp.dot(uref[...], w1u_ref[...], preferred_element_type=jnp.float32)
    h1 = h1 + jnp.dot(aref[...], w1a_ref[...],
                      preferred_element_type=jnp.float32)
    h1 = jnp.maximum(h1 + b1_ref[...], 0.0)
    h2 = jnp.dot(h1, w2_ref[...], preferred_element_type=jnp.float32)
    h2 = jnp.maximum(h2 + b2_ref[...], 0.0)
    out_ref[...] = jnp.sum(h2 * w3_ref[...], axis=1)


def _mlp(user_vec, anime_vec, W1, b1, W2, b2, W3, block_b):
    batch = user_vec.shape[0]
    grid = (batch // block_b,)
    full = lambda i: (0, 0)
    out = pl.pallas_call(
        _mlp_body,
        grid=grid,
        in_specs=[
            pl.BlockSpec((block_b, EMBED_DIM), lambda i: (i, 0)),
            pl.BlockSpec((block_b, EMBED_DIM), lambda i: (i, 0)),
            pl.BlockSpec((EMBED_DIM, 128), full),
            pl.BlockSpec((EMBED_DIM, 128), full),
            pl.BlockSpec((1, 128), full),
            pl.BlockSpec((128, EMBED_DIM), full),
            pl.BlockSpec((1, EMBED_DIM), full),
            pl.BlockSpec((1, EMBED_DIM), full),
        ],
        out_specs=pl.BlockSpec((block_b,), lambda i: (i,)),
        out_shape=jax.ShapeDtypeStruct((batch,), jnp.float32),
    )(user_vec, anime_vec, W1[:EMBED_DIM], W1[EMBED_DIM:],
      b1.reshape(1, 128), W2, b2.reshape(1, EMBED_DIM),
      W3.reshape(1, EMBED_DIM))
    return out


def kernel(user_id, anime_id, user_table, anime_table, W1, b1, W2, b2, W3, b3):
    batch = user_id.shape[0]
    num_workers = 32
    chunks_per_worker = batch // (num_workers * IDX_CHUNK)
    gk = _make_gather_kernel(batch, num_workers, chunks_per_worker)
    uidx2d = user_id.reshape(-1, IDX_CHUNK)
    aidx2d = anime_id.reshape(-1, IDX_CHUNK)
    user_vec, anime_vec = gk(uidx2d, aidx2d, user_table, anime_table)
    out = _mlp(user_vec, anime_vec, W1, b1, W2, b2, W3, block_b=2048)
    return out[:, None] + b3


# speedup vs baseline: 1.0051x; 1.0051x over previous
import functools

import jax
import jax.numpy as jnp
from jax import lax
from jax.experimental import pallas as pl
from jax.experimental.pallas import tpu as pltpu
from jax.experimental.pallas import tpu_sc as plsc

EMBED_DIM = 64
PAIR_DIM = 2 * EMBED_DIM
IDX_CHUNK = 128


def _make_gather_kernel(batch, num_workers, chunks_per_worker):
    rows_per_worker = chunks_per_worker * IDX_CHUNK
    mesh = plsc.VectorSubcoreMesh(core_axis_name="c", subcore_axis_name="s")

    @functools.partial(
        pl.kernel,
        out_type=(
            jax.ShapeDtypeStruct((batch, PAIR_DIM), jnp.float32),
            jax.ShapeDtypeStruct((batch, PAIR_DIM), jnp.float32),
        ),
        mesh=mesh,
        scratch_types=[
            pltpu.VMEM((chunks_per_worker, IDX_CHUNK), jnp.int32),
            pltpu.VMEM((chunks_per_worker, IDX_CHUNK), jnp.int32),
            pltpu.VMEM((rows_per_worker, PAIR_DIM), jnp.float32),
            pltpu.SemaphoreType.DMA,
        ],
    )
    def gather_kernel(uidx_hbm, aidx_hbm, utab_hbm, atab_hbm,
                      uout_hbm, aout_hbm,
                      uidx_v, aidx_v, rows_v, sem):
        wid = lax.axis_index("s") * 2 + lax.axis_index("c")
        crow = wid * chunks_per_worker
        base = wid * rows_per_worker
        pltpu.sync_copy(uidx_hbm.at[pl.ds(crow, chunks_per_worker)], uidx_v)
        pltpu.sync_copy(aidx_hbm.at[pl.ds(crow, chunks_per_worker)], aidx_v)
        copies = []
        for j in range(chunks_per_worker):
            copies.append(pltpu.async_copy(
                utab_hbm.at[uidx_v.at[j]],
                rows_v.at[pl.ds(j * IDX_CHUNK, IDX_CHUNK)], sem))
        for c in copies:
            c.wait()
        pltpu.sync_copy(rows_v, uout_hbm.at[pl.ds(base, rows_per_worker)])
        copies = []
        for j in range(chunks_per_worker):
            copies.append(pltpu.async_copy(
                atab_hbm.at[aidx_v.at[j]],
                rows_v.at[pl.ds(j * IDX_CHUNK, IDX_CHUNK)], sem))
        for c in copies:
            c.wait()
        pltpu.sync_copy(rows_v, aout_hbm.at[pl.ds(base, rows_per_worker)])

    return gather_kernel


def _mlp_body(upair_ref, apair_ref, uid_ref, aid_ref,
              w1u_ref, w1a_ref, b1_ref, w2_ref, b2_ref, w3_ref, out_ref):
    up = upair_ref[...]
    ap = apair_ref[...]
    usel = (uid_ref[...] & 1) == 1
    asel = (aid_ref[...] & 1) == 1
    u = jnp.where(usel, up[:, EMBED_DIM:], up[:, :EMBED_DIM])
    a = jnp.where(asel, ap[:, EMBED_DIM:], ap[:, :EMBED_DIM])
    h1 = jnp.dot(u, w1u_ref[...], preferred_element_type=jnp.float32)
    h1 = h1 + jnp.dot(a, w1a_ref[...], preferred_element_type=jnp.float32)
    h1 = jnp.maximum(h1 + b1_ref[...], 0.0)
    h2 = jnp.dot(h1, w2_ref[...], preferred_element_type=jnp.float32)
    h2 = jnp.maximum(h2 + b2_ref[...], 0.0)
    out_ref[...] = jnp.sum(h2 * w3_ref[...], axis=1)


def _mlp(upairs, apairs, user_id, anime_id, W1, b1, W2, b2, W3, block_b):
    batch = upairs.shape[0]
    grid = (batch // block_b,)
    full = lambda i: (0, 0)
    out = pl.pallas_call(
        _mlp_body,
        grid=grid,
        in_specs=[
            pl.BlockSpec((block_b, PAIR_DIM), lambda i: (i, 0)),
            pl.BlockSpec((block_b, PAIR_DIM), lambda i: (i, 0)),
            pl.BlockSpec((block_b, 1), lambda i: (i, 0)),
            pl.BlockSpec((block_b, 1), lambda i: (i, 0)),
            pl.BlockSpec((EMBED_DIM, 128), full),
            pl.BlockSpec((EMBED_DIM, 128), full),
            pl.BlockSpec((1, 128), full),
            pl.BlockSpec((128, EMBED_DIM), full),
            pl.BlockSpec((1, EMBED_DIM), full),
            pl.BlockSpec((1, EMBED_DIM), full),
        ],
        out_specs=pl.BlockSpec((block_b,), lambda i: (i,)),
        out_shape=jax.ShapeDtypeStruct((batch,), jnp.float32),
    )(upairs, apairs, user_id[:, None], anime_id[:, None],
      W1[:EMBED_DIM], W1[EMBED_DIM:],
      b1.reshape(1, 128), W2, b2.reshape(1, EMBED_DIM),
      W3.reshape(1, EMBED_DIM))
    return out


def kernel(user_id, anime_id, user_table, anime_table, W1, b1, W2, b2, W3, b3):
    batch = user_id.shape[0]
    num_workers = 32
    chunks_per_worker = batch // (num_workers * IDX_CHUNK)
    gk = _make_gather_kernel(batch, num_workers, chunks_per_worker)
    utab2 = user_table.reshape(-1, PAIR_DIM)
    atab2 = anime_table.reshape(-1, PAIR_DIM)
    uidx2d = (user_id // 2).reshape(-1, IDX_CHUNK)
    aidx2d = (anime_id // 2).reshape(-1, IDX_CHUNK)
    upairs, apairs = gk(uidx2d, aidx2d, utab2, atab2)
    out = _mlp(upairs, apairs, user_id, anime_id,
               W1, b1, W2, b2, W3, block_b=2048)
    return out[:, None] + b3
